# Initial kernel scaffold; baseline (speedup 1.0000x reference)
import jax
import jax.numpy as jnp
from jax.experimental import pallas as pl


def kernel(query, prompt_pool, prompt_key):
    raise NotImplementedError("write your pallas kernel here")


# speedup vs baseline: 1.8779x; 1.8779x over previous
import functools

import jax
import jax.numpy as jnp
from jax import lax
from jax.experimental import pallas as pl
from jax.experimental.pallas import tpu as pltpu
from jax.experimental.pallas import tpu_sc as plsc

_B = 1024
_D = 768
_P = 8192
_K = 5
_PLEN = 4
_PB = 1024
_ROWD = _PLEN * _D

_NC = 2
_NS = 16
_NW = _NC * _NS
_BFLAT = _B * _K
_BPW = _BFLAT // _NW
_CH = 32
_NCH = _BPW // _CH


def _norm_rows(x):
    n = jnp.sqrt(jnp.sum(x * x, axis=1, keepdims=True))
    return x / jnp.maximum(n, 1e-12)


def _tc_body(q_ref, k_ref, sim_ref, tkv_ref, tki_ref):
    j = pl.program_id(0)

    @pl.when(j == 0)
    def _init():
        tkv_ref[...] = jnp.full((_B, 128), -jnp.inf, dtype=jnp.float32)
        tki_ref[...] = jnp.zeros((_B, 128), dtype=jnp.int32)

    s = lax.dot_general(
        q_ref[...], k_ref[...],
        dimension_numbers=(((1,), (1,)), ((), ())),
        preferred_element_type=jnp.float32,
        precision=lax.Precision.DEFAULT,
    )
    sim_ref[...] = s

    gidx = lax.broadcasted_iota(jnp.int32, (_B, _PB), 1) + j * _PB
    cand_v = jnp.concatenate([s, tkv_ref[...]], axis=1)
    cand_i = jnp.concatenate([gidx, tki_ref[...]], axis=1)

    cols = lax.broadcasted_iota(jnp.int32, (_B, 128), 1)
    acc_v = jnp.full((_B, 128), -jnp.inf, dtype=jnp.float32)
    acc_i = jnp.zeros((_B, 128), dtype=jnp.int32)
    for t in range(_K):
        m = jnp.max(cand_v, axis=1, keepdims=True)
        gi = jnp.min(jnp.where(cand_v == m, cand_i, jnp.int32(2**31 - 1)),
                     axis=1, keepdims=True)
        acc_v = jnp.where(cols == t, m, acc_v)
        acc_i = jnp.where(cols == t, gi, acc_i)
        cand_v = jnp.where(cand_i == gi, -jnp.inf, cand_v)
    tkv_ref[...] = acc_v
    tki_ref[...] = acc_i


_tc_call = pl.pallas_call(
    _tc_body,
    grid=(_P // _PB,),
    in_specs=[
        pl.BlockSpec((_B, _D), lambda j: (0, 0)),
        pl.BlockSpec((_PB, _D), lambda j: (j, 0)),
    ],
    out_specs=[
        pl.BlockSpec((_B, _PB), lambda j: (0, j)),
        pl.BlockSpec((_B, 128), lambda j: (0, 0)),
        pl.BlockSpec((_B, 128), lambda j: (0, 0)),
    ],
    out_shape=[
        jax.ShapeDtypeStruct((_B, _P), jnp.float32),
        jax.ShapeDtypeStruct((_B, 128), jnp.float32),
        jax.ShapeDtypeStruct((_B, 128), jnp.int32),
    ],
    compiler_params=pltpu.CompilerParams(
        dimension_semantics=("arbitrary",),
    ),
)


def _sc_gather_body(table_hbm, idx_hbm, out_hbm, idx_v, rows_v, sem):
    wid = lax.axis_index("s") * _NC + lax.axis_index("c")
    base = wid * _BPW
    pltpu.sync_copy(idx_hbm.at[pl.ds(base, _BPW)], idx_v)
    for c in range(_NCH):
        pltpu.async_copy(
            table_hbm.at[idx_v.at[pl.ds(c * _CH, _CH)]], rows_v, sem
        ).wait()
        pltpu.sync_copy(rows_v, out_hbm.at[pl.ds(base + c * _CH, _CH)])


@functools.cache
def _sc_gather():
    return pl.kernel(
        _sc_gather_body,
        out_type=jax.ShapeDtypeStruct((_BFLAT, _ROWD), jnp.float32),
        mesh=plsc.VectorSubcoreMesh(core_axis_name="c", subcore_axis_name="s"),
        scratch_types=[
            pltpu.VMEM((_BPW,), jnp.int32),
            pltpu.VMEM((_CH, _ROWD), jnp.float32),
            pltpu.SemaphoreType.DMA,
        ],
    )


def kernel(query, prompt_pool, prompt_key):
    qn = _norm_rows(query)
    kn = _norm_rows(prompt_key)
    sim, tkv, tki = _tc_call(qn, kn)
    top_k_similarity = tkv[:, :_K]
    idx_flat = tki[:, :_K].reshape(-1)
    table = prompt_pool.reshape(_P, _ROWD)
    gathered = _sc_gather()(table, idx_flat)
    selected_prompts = gathered.reshape(_B, _K * _PLEN, _D)
    return selected_prompts, sim, top_k_similarity


# speedup vs baseline: 3.0211x; 1.6087x over previous
import functools

import jax
import jax.numpy as jnp
from jax import lax
from jax.experimental import pallas as pl
from jax.experimental.pallas import tpu as pltpu
from jax.experimental.pallas import tpu_sc as plsc

_B = 1024
_D = 768
_P = 8192
_K = 5
_PLEN = 4
_PB = 1024
_ROWD = _PLEN * _D

_NC = 2
_NS = 16
_NW = _NC * _NS
_BFLAT = _B * _K
_BPW = _BFLAT // _NW
_CH = 32
_NCH = _BPW // _CH


def _norm_rows(x):
    n = jnp.sqrt(jnp.sum(x * x, axis=1, keepdims=True))
    return x / jnp.maximum(n, 1e-12)


def _tc_body(q_ref, k_ref, sim_ref, tkv_ref, tki_ref):
    j = pl.program_id(0)

    @pl.when(j == 0)
    def _init():
        tkv_ref[...] = jnp.full((_B, 128), -jnp.inf, dtype=jnp.float32)
        tki_ref[...] = jnp.zeros((_B, 128), dtype=jnp.int32)

    s = lax.dot_general(
        q_ref[...], k_ref[...],
        dimension_numbers=(((1,), (1,)), ((), ())),
        preferred_element_type=jnp.float32,
        precision=lax.Precision.DEFAULT,
    )
    sim_ref[...] = s

    gidx = lax.broadcasted_iota(jnp.int32, (_B, _PB), 1) + j * _PB
    cand_v = jnp.concatenate([s, tkv_ref[...]], axis=1)
    cand_i = jnp.concatenate([gidx, tki_ref[...]], axis=1)

    cols = lax.broadcasted_iota(jnp.int32, (_B, 128), 1)
    acc_v = jnp.full((_B, 128), -jnp.inf, dtype=jnp.float32)
    acc_i = jnp.zeros((_B, 128), dtype=jnp.int32)
    for t in range(_K):
        m = jnp.max(cand_v, axis=1, keepdims=True)
        gi = jnp.min(jnp.where(cand_v == m, cand_i, jnp.int32(2**31 - 1)),
                     axis=1, keepdims=True)
        acc_v = jnp.where(cols == t, m, acc_v)
        acc_i = jnp.where(cols == t, gi, acc_i)
        cand_v = jnp.where(cand_i == gi, -jnp.inf, cand_v)
    tkv_ref[...] = acc_v
    tki_ref[...] = acc_i


_tc_call = pl.pallas_call(
    _tc_body,
    grid=(_P // _PB,),
    in_specs=[
        pl.BlockSpec((_B, _D), lambda j: (0, 0)),
        pl.BlockSpec((_PB, _D), lambda j: (j, 0)),
    ],
    out_specs=[
        pl.BlockSpec((_B, _PB), lambda j: (0, j)),
        pl.BlockSpec((_B, 128), lambda j: (0, 0)),
        pl.BlockSpec((_B, 128), lambda j: (0, 0)),
    ],
    out_shape=[
        jax.ShapeDtypeStruct((_B, _P), jnp.float32),
        jax.ShapeDtypeStruct((_B, 128), jnp.float32),
        jax.ShapeDtypeStruct((_B, 128), jnp.int32),
    ],
    compiler_params=pltpu.CompilerParams(
        dimension_semantics=("arbitrary",),
    ),
)


def _sc_gather_body(table_hbm, idx_hbm, out_hbm, idx_v, rows_v, sem):
    wid = lax.axis_index("s") * _NC + lax.axis_index("c")
    base = wid * _BPW
    b0 = wid * (_BPW // _K)
    pltpu.sync_copy(idx_hbm.at[pl.ds(base, _BPW)], idx_v)
    for c in range(_NCH):
        pltpu.async_copy(
            table_hbm.at[idx_v.at[pl.ds(c * _CH, _CH)]], rows_v, sem
        ).wait()
        for i in range(_CH):
            r = c * _CH + i
            pltpu.sync_copy(
                rows_v.at[i],
                out_hbm.at[b0 + r // _K].at[pl.ds(_PLEN * (r % _K), _PLEN)],
            )


@functools.cache
def _sc_gather():
    return pl.kernel(
        _sc_gather_body,
        out_type=jax.ShapeDtypeStruct((_B, _K * _PLEN, _D), jnp.float32),
        mesh=plsc.VectorSubcoreMesh(core_axis_name="c", subcore_axis_name="s"),
        scratch_types=[
            pltpu.VMEM((_BPW,), jnp.int32),
            pltpu.VMEM((_CH, _PLEN, _D), jnp.float32),
            pltpu.SemaphoreType.DMA,
        ],
        compiler_params=pltpu.CompilerParams(use_tc_tiling_on_sc=True),
    )


def kernel(query, prompt_pool, prompt_key):
    qn = _norm_rows(query)
    kn = _norm_rows(prompt_key)
    sim, tkv, tki = _tc_call(qn, kn)
    top_k_similarity = tkv[:, :_K]
    idx_flat = tki[:, :_K].reshape(-1)
    selected_prompts = _sc_gather()(prompt_pool, idx_flat)
    return selected_prompts, sim, top_k_similarity
